# baseline (speedup 1.0000x reference)
import functools

import jax
import jax.numpy as jnp
from jax import lax
from jax.experimental import pallas as pl
from jax.experimental.pallas import tpu as pltpu
from jax.experimental.pallas import tpu_sc as plsc

_NUM_CORES = 2
_NUM_SUBCORES = 16
_NUM_WORKERS = _NUM_CORES * _NUM_SUBCORES

_GROUP = 16
_NBUF = 4


def _sc_row_gather(x_flat, wt):
    n = x_flat.shape[0]
    vocab, d_model = wt.shape
    j_per_w = n // _NUM_WORKERS
    groups = j_per_w // _GROUP

    mesh = plsc.VectorSubcoreMesh(core_axis_name="c", subcore_axis_name="s")

    @functools.partial(
        pl.kernel,
        out_type=jax.ShapeDtypeStruct((n, d_model), jnp.float32),
        mesh=mesh,
        scratch_types=(
            [pltpu.VMEM((_GROUP,), jnp.int32) for _ in range(_NBUF)]
            + [pltpu.VMEM((_GROUP, d_model), jnp.float32)
               for _ in range(_NBUF)]
            + [pltpu.SemaphoreType.DMA((_NBUF,)),
               pltpu.SemaphoreType.DMA((_NBUF,))]
        ),
        compiler_params=pltpu.CompilerParams(needs_layout_passes=False),
    )
    def sc_kernel(x_hbm, w_hbm, out_hbm, *bufs):
        ibufs = list(bufs[:_NBUF])
        gbufs = list(bufs[_NBUF:2 * _NBUF])
        gsem, ssem = bufs[2 * _NBUF], bufs[2 * _NBUF + 1]
        wid = lax.axis_index("s") * _NUM_CORES + lax.axis_index("c")

        def load_idx(g, b):
            pltpu.sync_copy(
                x_hbm.at[pl.ds(wid * j_per_w + g * _GROUP, _GROUP)], ibufs[b])

        def fire_gather(b):
            pltpu.async_copy(w_hbm.at[ibufs[b]], gbufs[b], gsem.at[b])

        def wait_gather(b):
            pltpu.make_async_copy(w_hbm.at[ibufs[b]], gbufs[b],
                                  gsem.at[b]).wait()

        def out_slice(g):
            return out_hbm.at[pl.ds(wid * j_per_w + g * _GROUP, _GROUP), :]

        def fire_scatter(g, b):
            pltpu.async_copy(gbufs[b], out_slice(g), ssem.at[b])

        def wait_scatter(g, b):
            pltpu.make_async_copy(gbufs[b], out_slice(g), ssem.at[b]).wait()

        for b in range(_NBUF):
            load_idx(b, b)
            fire_gather(b)

        def steady(gouter, carry):
            for b in range(_NBUF):
                g = gouter * _NBUF + b
                wait_gather(b)
                fire_scatter(g, b)

                @pl.when(g + _NBUF < groups)
                def _():
                    load_idx(g + _NBUF, b)
                    wait_scatter(g, b)
                    fire_gather(b)

                @pl.when(g + _NBUF >= groups)
                def _():
                    wait_scatter(g, b)
            return carry

        lax.fori_loop(0, groups // _NBUF, steady, 0)

    return sc_kernel(x_flat, wt)


def kernel(x, W_embed):
    b, p = x.shape
    d_model, vocab = W_embed.shape
    n = b * p
    x_flat = x.reshape(n).astype(jnp.int32)
    wt = W_embed.T.reshape(vocab, d_model)
    out = _sc_row_gather(x_flat, wt)
    return out.reshape(b, p, d_model)


# speedup vs baseline: 1.0787x; 1.0787x over previous
import functools

import jax
import jax.numpy as jnp
from jax import lax
from jax.experimental import pallas as pl
from jax.experimental.pallas import tpu as pltpu
from jax.experimental.pallas import tpu_sc as plsc

_NUM_CORES = 2
_NUM_SUBCORES = 16
_NUM_WORKERS = _NUM_CORES * _NUM_SUBCORES

_GROUP = 32
_NBUF = 3


def _sc_row_gather(x_flat, wt):
    n = x_flat.shape[0]
    vocab, d_model = wt.shape
    j_per_w = n // _NUM_WORKERS
    groups = j_per_w // _GROUP

    mesh = plsc.VectorSubcoreMesh(core_axis_name="c", subcore_axis_name="s")

    @functools.partial(
        pl.kernel,
        out_type=jax.ShapeDtypeStruct((n, d_model), jnp.float32),
        mesh=mesh,
        scratch_types=(
            [pltpu.VMEM((_GROUP,), jnp.int32) for _ in range(_NBUF)]
            + [pltpu.VMEM((_GROUP, d_model), jnp.float32)
               for _ in range(_NBUF)]
            + [pltpu.SemaphoreType.DMA((_NBUF,)),
               pltpu.SemaphoreType.DMA((_NBUF,))]
        ),
        compiler_params=pltpu.CompilerParams(needs_layout_passes=False),
    )
    def sc_kernel(x_hbm, w_hbm, out_hbm, *bufs):
        ibufs = list(bufs[:_NBUF])
        gbufs = list(bufs[_NBUF:2 * _NBUF])
        gsem, ssem = bufs[2 * _NBUF], bufs[2 * _NBUF + 1]
        wid = lax.axis_index("s") * _NUM_CORES + lax.axis_index("c")

        def load_idx(g, b):
            pltpu.sync_copy(
                x_hbm.at[pl.ds(wid * j_per_w + g * _GROUP, _GROUP)], ibufs[b])

        def fire_gather(b):
            pltpu.async_copy(w_hbm.at[ibufs[b]], gbufs[b], gsem.at[b])

        def wait_gather(b):
            pltpu.make_async_copy(w_hbm.at[ibufs[b]], gbufs[b],
                                  gsem.at[b]).wait()

        def out_slice(g):
            return out_hbm.at[pl.ds(wid * j_per_w + g * _GROUP, _GROUP), :]

        def fire_scatter(g, b):
            pltpu.async_copy(gbufs[b], out_slice(g), ssem.at[b])

        def wait_scatter(g, b):
            pltpu.make_async_copy(gbufs[b], out_slice(g), ssem.at[b]).wait()

        for b in range(_NBUF):
            load_idx(b, b)
            fire_gather(b)

        def steady(gouter, carry):
            for b in range(_NBUF):
                g = gouter * _NBUF + b
                wait_gather(b)
                fire_scatter(g, b)

                @pl.when(g + _NBUF < groups)
                def _():
                    load_idx(g + _NBUF, b)
                    wait_scatter(g, b)
                    fire_gather(b)

                @pl.when(g + _NBUF >= groups)
                def _():
                    wait_scatter(g, b)
            return carry

        lax.fori_loop(0, groups // _NBUF, steady, 0)

        for b in range(groups % _NBUF):
            g = (groups // _NBUF) * _NBUF + b
            wait_gather(b)
            fire_scatter(g, b)
            wait_scatter(g, b)

    return sc_kernel(x_flat, wt)


def kernel(x, W_embed):
    b, p = x.shape
    d_model, vocab = W_embed.shape
    n = b * p
    x_flat = x.reshape(n).astype(jnp.int32)
    wt = W_embed.T.reshape(vocab, d_model)
    out = _sc_row_gather(x_flat, wt)
    return out.reshape(b, p, d_model)
